# baseline (speedup 1.0000x reference)
import functools

import jax
import jax.numpy as jnp
from jax import lax
from jax.experimental import pallas as pl
from jax.experimental.pallas import tpu as pltpu
from jax.experimental.pallas import tpu_sc as plsc

N = 50000
NACC = 50016
E = 800000
EP = 802816
EROWS = EP // 128
VOCAB = 100000
EMB = 32
NPROP = 3
F = 96
G3 = 288
C2 = 20
NSTEP = 5
SMIN = 2
NCHUNK = 3
NC, NS, NW = 2, 16, 32
EJ = EROWS // NW
GK = 28
NBUF = 4
EJ0, NQ0 = 168, 6
EJ1, NQ1 = 224, 8
_SMALL_CORE = 1
STRIPE = NACC // NS
IDXP = 163840
IROWS = IDXP // 128
IJ = IROWS // NW
NBUF = 4

f32 = jnp.float32
i32 = jnp.int32
_DN = (((1,), (1,)), ((), ()))
_PREC = None

_mesh = plsc.VectorSubcoreMesh(core_axis_name="c", subcore_axis_name="s")
_SC_PARAMS = pltpu.CompilerParams(use_tc_tiling_on_sc=False)


def _embed_body(table_hbm, idx_hbm, out_hbm, idx_v, e0, e1, e2, e3,
                s0, s1, s2, s3):
    ebuf = [e0, e1, e2, e3]
    sem = [s0, s1, s2, s3]
    wid = lax.axis_index("c") * NS + lax.axis_index("s")
    rbase = wid * IJ
    pltpu.sync_copy(idx_hbm.at[pl.ds(rbase, IJ)], idx_v)
    for b in range(NBUF):
        pltpu.async_copy(table_hbm.at[idx_v.at[b]], ebuf[b], sem[b])

    @pl.loop(0, IJ // NBUF)
    def _grp(g):
        for b in range(NBUF):
            j = g * NBUF + b
            pltpu.make_async_copy(table_hbm.at[idx_v.at[j]], ebuf[b],
                                  sem[b]).wait()
            pltpu.sync_copy(ebuf[b], out_hbm.at[pl.ds((rbase + j) * 128, 128)])

            @pl.when(j + NBUF < IJ)
            def _():
                pltpu.async_copy(table_hbm.at[idx_v.at[j + NBUF]], ebuf[b],
                                 sem[b])


_embed_call = pl.kernel(
    _embed_body,
    out_type=jax.ShapeDtypeStruct((IDXP, EMB), f32),
    mesh=_mesh,
    compiler_params=_SC_PARAMS,
    scratch_types=[
        pltpu.VMEM((IJ, 128), i32),
        pltpu.VMEM((128, EMB), f32),
        pltpu.VMEM((128, EMB), f32),
        pltpu.VMEM((128, EMB), f32),
        pltpu.VMEM((128, EMB), f32),
        pltpu.SemaphoreType.DMA,
        pltpu.SemaphoreType.DMA,
        pltpu.SemaphoreType.DMA,
        pltpu.SemaphoreType.DMA,
    ],
)


def _scatter_body(src_hbm, dst_hbm, m0, m1, m2, zero_hbm, out_hbm,
                  src_v, dst_v, e0, e1, e2, e3,
                  isem_s, isem_d, g0, g1, g2, g3, acc):
    ebuf = [e0, e1, e2, e3]
    gsem = [g0, g1, g2, g3]
    core = lax.axis_index("c")
    sub = lax.axis_index("s")
    is_big = core != _SMALL_CORE
    rbase = jnp.where(is_big, NS * EJ0 + sub * EJ1, sub * EJ0)

    def fire_idx(q, p):
        pltpu.async_copy(src_hbm.at[pl.ds(rbase + q * GK, GK)], src_v.at[p],
                         isem_s.at[p])
        pltpu.async_copy(dst_hbm.at[pl.ds(rbase + q * GK, GK)], dst_v.at[p],
                         isem_d.at[p])

    def wait_idx(q, p):
        pltpu.make_async_copy(src_hbm.at[pl.ds(rbase + q * GK, GK)],
                              src_v.at[p], isem_s.at[p]).wait()
        pltpu.make_async_copy(dst_hbm.at[pl.ds(rbase + q * GK, GK)],
                              dst_v.at[p], isem_d.at[p]).wait()

    def pipe(p, mc):
        for b in range(NBUF):
            pltpu.async_copy(mc.at[src_v.at[p, b]], ebuf[b], gsem[b])

        @pl.loop(0, GK // NBUF)
        def _grp(g, p=p, mc=mc):
            for b in range(NBUF):
                j = g * NBUF + b
                pltpu.make_async_copy(mc.at[src_v.at[p, j]], ebuf[b],
                                      gsem[b]).wait()
                pltpu.sync_copy(ebuf[b], acc.at[dst_v.at[p, j]], add=True)

                @pl.when(j + NBUF < GK)
                def _(p=p, mc=mc, j=j, b=b):
                    pltpu.async_copy(mc.at[src_v.at[p, j + NBUF]],
                                     ebuf[b], gsem[b])

    fire_idx(0, 0)
    for c, mc in enumerate((m0, m1, m2)):
        pltpu.sync_copy(zero_hbm, acc.at[pl.ds(sub * STRIPE, STRIPE)])
        plsc.subcore_barrier()
        for q in range(NQ1):
            p = q % 2
            if q < NQ0:
                wait_idx(q, p)
                if q + 1 < NQ0:
                    fire_idx(q + 1, 1 - p)
                else:
                    @pl.when(is_big)
                    def _(q=q, p=p):
                        fire_idx(q + 1, 1 - p)

                    if c + 1 < NCHUNK:
                        @pl.when(jnp.logical_not(is_big))
                        def _(p=p):
                            fire_idx(0, 1 - p)
                pipe(p, mc)
            else:
                @pl.when(is_big)
                def _(q=q, p=p, mc=mc, c=c):
                    wait_idx(q, p)
                    if q + 1 < NQ1:
                        fire_idx(q + 1, 1 - p)
                    elif c + 1 < NCHUNK:
                        fire_idx(0, 1 - p)
                    pipe(p, mc)

        plsc.subcore_barrier()
        pltpu.sync_copy(acc.at[pl.ds(sub * STRIPE, STRIPE)],
                        out_hbm.at[core, c, pl.ds(sub * STRIPE, STRIPE)])


_scatter_call = pl.kernel(
    _scatter_body,
    out_type=jax.ShapeDtypeStruct((NC, NCHUNK, NACC, EMB), f32),
    mesh=_mesh,
    compiler_params=_SC_PARAMS,
    scratch_types=[
        pltpu.VMEM((2, GK, 128), i32),
        pltpu.VMEM((2, GK, 128), i32),
        pltpu.VMEM((128, EMB), f32),
        pltpu.VMEM((128, EMB), f32),
        pltpu.VMEM((128, EMB), f32),
        pltpu.VMEM((128, EMB), f32),
        pltpu.SemaphoreType.DMA((2,)),
        pltpu.SemaphoreType.DMA((2,)),
        pltpu.SemaphoreType.DMA,
        pltpu.SemaphoreType.DMA,
        pltpu.SemaphoreType.DMA,
        pltpu.SemaphoreType.DMA,
        pltpu.VMEM_SHARED((NACC, EMB), f32),
    ],
)

NB = 2000
GRID = N // NB


def _write_m_chunks(hn, we, m_refs):
    for c, mref in enumerate(m_refs):
        mref[...] = lax.dot_general(hn, we[c * EMB:(c + 1) * EMB, :], _DN,
                                    precision=_PREC,
                                    preferred_element_type=f32)


def _tc_m0_body(h_ref, we_ref, m0_ref, m1_ref, m2_ref):
    _write_m_chunks(h_ref[...], we_ref[...], (m0_ref, m1_ref, m2_ref))


_m_out_shapes = [jax.ShapeDtypeStruct((N, EMB), f32) for _ in range(NCHUNK)]
_m_out_specs = [pl.BlockSpec((NB, EMB), lambda i: (i, 0)) for _ in range(NCHUNK)]

_tc_m0 = pl.pallas_call(
    _tc_m0_body,
    grid=(GRID,),
    in_specs=[
        pl.BlockSpec((NB, F), lambda i: (i, 0)),
        pl.BlockSpec((F, F), lambda i: (0, 0)),
    ],
    out_specs=_m_out_specs,
    out_shape=_m_out_shapes,
)


def _gru(h, p_ref, wih_ref, whh_ref, bih_ref, bhh_ref):
    p = p_ref[...]
    a = jnp.concatenate(
        [p[0, 0] + p[1, 0], p[0, 1] + p[1, 1], p[0, 2] + p[1, 2]], axis=1)
    gi = lax.dot_general(a, wih_ref[...], _DN, precision=_PREC,
                         preferred_element_type=f32) + bih_ref[...]
    gh = lax.dot_general(h, whh_ref[...], _DN, precision=_PREC,
                         preferred_element_type=f32) + bhh_ref[...]
    r = jax.nn.sigmoid(gi[:, :F] + gh[:, :F])
    z = jax.nn.sigmoid(gi[:, F:2 * F] + gh[:, F:2 * F])
    n = jnp.tanh(gi[:, 2 * F:] + r * gh[:, 2 * F:])
    return (1.0 - z) * n + z * h


def _tc_step_body(h_ref, p_ref, wih_ref, whh_ref, bih_ref, bhh_ref, we_ref,
                  hn_ref, m0_ref, m1_ref, m2_ref):
    hn = _gru(h_ref[...], p_ref, wih_ref, whh_ref, bih_ref, bhh_ref)
    hn_ref[...] = hn
    _write_m_chunks(hn, we_ref[...], (m0_ref, m1_ref, m2_ref))


_gru_in_specs = [
    pl.BlockSpec((NB, F), lambda i: (i, 0)),
    pl.BlockSpec((NC, NCHUNK, NB, EMB), lambda i: (0, 0, i, 0)),
    pl.BlockSpec((G3, F), lambda i: (0, 0)),
    pl.BlockSpec((G3, F), lambda i: (0, 0)),
    pl.BlockSpec((1, G3), lambda i: (0, 0)),
    pl.BlockSpec((1, G3), lambda i: (0, 0)),
]

_tc_step = pl.pallas_call(
    _tc_step_body,
    grid=(GRID,),
    in_specs=[
        *_gru_in_specs,
        pl.BlockSpec((F, F), lambda i: (0, 0)),
    ],
    out_specs=[
        pl.BlockSpec((NB, F), lambda i: (i, 0)),
        *_m_out_specs,
    ],
    out_shape=[
        jax.ShapeDtypeStruct((N, F), f32),
        *_m_out_shapes,
    ],
)


def _tc_final_body(h_ref, p_ref, wih_ref, whh_ref, bih_ref, bhh_ref,
                   h2_ref, h3_ref, cw_ref, cb_ref, out_ref):
    h4 = h_ref[...]
    h5 = _gru(h4, p_ref, wih_ref, whh_ref, bih_ref, bhh_ref)
    cw = cw_ref[...]
    cb = cb_ref[...]
    cs = [
        jax.nn.sigmoid(
            lax.dot_general(hh, cw, _DN, precision=_PREC,
                            preferred_element_type=f32) + cb)
        for hh in (h2_ref[...], h3_ref[...], h4, h5)
    ]
    out_ref[...] = jnp.stack(cs, axis=1).reshape(4 * NB, C2)


_tc_final = pl.pallas_call(
    _tc_final_body,
    grid=(GRID,),
    in_specs=[
        *_gru_in_specs,
        pl.BlockSpec((NB, F), lambda i: (i, 0)),
        pl.BlockSpec((NB, F), lambda i: (i, 0)),
        pl.BlockSpec((C2, F), lambda i: (0, 0)),
        pl.BlockSpec((1, C2), lambda i: (0, 0)),
    ],
    out_specs=pl.BlockSpec((4 * NB, C2), lambda i: (i, 0)),
    out_shape=jax.ShapeDtypeStruct((4 * N, C2), f32),
)


def kernel(prop_ids, edge_index, step_min, step_max, embed_table, W_e, w_ih,
           w_hh, b_ih, b_hh, conv_w, conv_b):
    del step_min, step_max
    idx = prop_ids.astype(i32).reshape(-1)
    idx = jnp.concatenate([idx, jnp.zeros((IDXP - N * NPROP,), i32)])
    idx2 = idx.reshape(IROWS, 128)
    emb = _embed_call(embed_table, idx2)
    h = emb[:N * NPROP].reshape(N, F)

    src = edge_index[0].astype(i32)
    dst = edge_index[1].astype(i32)
    pad_e = EP - E
    src_p = jnp.concatenate([src, jnp.zeros((pad_e,), i32)]).reshape(EROWS, 128)
    dst_pad = N + (jnp.arange(pad_e, dtype=i32) % (NACC - N))
    dst_p = jnp.concatenate([dst, dst_pad]).reshape(EROWS, 128)
    zeros_blk = jnp.zeros((STRIPE, EMB), f32)

    bih2 = b_ih.reshape(1, G3)
    bhh2 = b_hh.reshape(1, G3)
    cb2 = conv_b.reshape(1, C2)

    m0, m1, m2 = _tc_m0(h, W_e)
    hs = [h]
    for _ in range(NSTEP - 1):
        part = _scatter_call(src_p, dst_p, m0, m1, m2, zeros_blk)
        h, m0, m1, m2 = _tc_step(h, part, w_ih, w_hh, bih2, bhh2, W_e)
        hs.append(h)
    part = _scatter_call(src_p, dst_p, m0, m1, m2, zeros_blk)
    out = _tc_final(h, part, w_ih, w_hh, bih2, bhh2, hs[2], hs[3],
                    conv_w, cb2)
    return out


# speedup vs baseline: 1.0535x; 1.0535x over previous
import functools

import jax
import jax.numpy as jnp
from jax import lax
from jax.experimental import pallas as pl
from jax.experimental.pallas import tpu as pltpu
from jax.experimental.pallas import tpu_sc as plsc

N = 50000
NACC = 50016
E = 800000
EP = 802816
EROWS = EP // 128
VOCAB = 100000
EMB = 32
NPROP = 3
F = 96
G3 = 288
C2 = 20
NSTEP = 5
SMIN = 2
NCHUNK = 3
NC, NS, NW = 2, 16, 32
EJ = EROWS // NW
GK = 28
NQ = EJ // GK
NBUF = 4
STRIPE = NACC // NS
IDXP = 163840
IROWS = IDXP // 128
IJ = IROWS // NW
NBUF = 4

f32 = jnp.float32
i32 = jnp.int32
_DN = (((1,), (1,)), ((), ()))
_PREC = None

_mesh = plsc.VectorSubcoreMesh(core_axis_name="c", subcore_axis_name="s")
_SC_PARAMS = pltpu.CompilerParams(use_tc_tiling_on_sc=False)


def _embed_body(table_hbm, idx_hbm, out_hbm, idx_v, e0, e1, e2, e3,
                s0, s1, s2, s3):
    ebuf = [e0, e1, e2, e3]
    sem = [s0, s1, s2, s3]
    wid = lax.axis_index("c") * NS + lax.axis_index("s")
    rbase = wid * IJ
    pltpu.sync_copy(idx_hbm.at[pl.ds(rbase, IJ)], idx_v)
    for b in range(NBUF):
        pltpu.async_copy(table_hbm.at[idx_v.at[b]], ebuf[b], sem[b])

    @pl.loop(0, IJ // NBUF)
    def _grp(g):
        for b in range(NBUF):
            j = g * NBUF + b
            pltpu.make_async_copy(table_hbm.at[idx_v.at[j]], ebuf[b],
                                  sem[b]).wait()
            pltpu.sync_copy(ebuf[b], out_hbm.at[pl.ds((rbase + j) * 128, 128)])

            @pl.when(j + NBUF < IJ)
            def _():
                pltpu.async_copy(table_hbm.at[idx_v.at[j + NBUF]], ebuf[b],
                                 sem[b])


_embed_call = pl.kernel(
    _embed_body,
    out_type=jax.ShapeDtypeStruct((IDXP, EMB), f32),
    mesh=_mesh,
    compiler_params=_SC_PARAMS,
    scratch_types=[
        pltpu.VMEM((IJ, 128), i32),
        pltpu.VMEM((128, EMB), f32),
        pltpu.VMEM((128, EMB), f32),
        pltpu.VMEM((128, EMB), f32),
        pltpu.VMEM((128, EMB), f32),
        pltpu.SemaphoreType.DMA,
        pltpu.SemaphoreType.DMA,
        pltpu.SemaphoreType.DMA,
        pltpu.SemaphoreType.DMA,
    ],
)


def _scatter_body(src_hbm, dst_hbm, m0, m1, m2, zero_hbm, out_hbm,
                  src_v, dst_v, e0, e1, e2, e3,
                  isem_s, isem_d, g0, g1, g2, g3, acc):
    ebuf = [e0, e1, e2, e3]
    gsem = [g0, g1, g2, g3]
    core = lax.axis_index("c")
    sub = lax.axis_index("s")
    rbase = (core * NS + sub) * EJ

    def fire_idx(q, p):
        pltpu.async_copy(src_hbm.at[pl.ds(rbase + q * GK, GK)], src_v.at[p],
                         isem_s.at[p])
        pltpu.async_copy(dst_hbm.at[pl.ds(rbase + q * GK, GK)], dst_v.at[p],
                         isem_d.at[p])

    def wait_idx(q, p):
        pltpu.make_async_copy(src_hbm.at[pl.ds(rbase + q * GK, GK)],
                              src_v.at[p], isem_s.at[p]).wait()
        pltpu.make_async_copy(dst_hbm.at[pl.ds(rbase + q * GK, GK)],
                              dst_v.at[p], isem_d.at[p]).wait()

    def pipe(p, mc):
        for b in range(NBUF):
            pltpu.async_copy(mc.at[src_v.at[p, b]], ebuf[b], gsem[b])

        @pl.loop(0, GK // NBUF)
        def _grp(g, p=p, mc=mc):
            for b in range(NBUF):
                j = g * NBUF + b
                pltpu.make_async_copy(mc.at[src_v.at[p, j]], ebuf[b],
                                      gsem[b]).wait()
                pltpu.sync_copy(ebuf[b], acc.at[dst_v.at[p, j]], add=True)

                @pl.when(j + NBUF < GK)
                def _(p=p, mc=mc, j=j, b=b):
                    pltpu.async_copy(mc.at[src_v.at[p, j + NBUF]],
                                     ebuf[b], gsem[b])

    fire_idx(0, 0)
    for c, mc in enumerate((m0, m1, m2)):
        pltpu.sync_copy(zero_hbm, acc.at[pl.ds(sub * STRIPE, STRIPE)])
        plsc.subcore_barrier()
        for q in range(NQ):
            p = (c * NQ + q) % 2
            wait_idx(q, p)
            if q + 1 < NQ:
                fire_idx(q + 1, 1 - p)
            elif c + 1 < NCHUNK:
                fire_idx(0, 1 - p)
            pipe(p, mc)

        plsc.subcore_barrier()
        pltpu.sync_copy(acc.at[pl.ds(sub * STRIPE, STRIPE)],
                        out_hbm.at[core, c, pl.ds(sub * STRIPE, STRIPE)])


_scatter_call = pl.kernel(
    _scatter_body,
    out_type=jax.ShapeDtypeStruct((NC, NCHUNK, NACC, EMB), f32),
    mesh=_mesh,
    compiler_params=_SC_PARAMS,
    scratch_types=[
        pltpu.VMEM((2, GK, 128), i32),
        pltpu.VMEM((2, GK, 128), i32),
        pltpu.VMEM((128, EMB), f32),
        pltpu.VMEM((128, EMB), f32),
        pltpu.VMEM((128, EMB), f32),
        pltpu.VMEM((128, EMB), f32),
        pltpu.SemaphoreType.DMA((2,)),
        pltpu.SemaphoreType.DMA((2,)),
        pltpu.SemaphoreType.DMA,
        pltpu.SemaphoreType.DMA,
        pltpu.SemaphoreType.DMA,
        pltpu.SemaphoreType.DMA,
        pltpu.VMEM_SHARED((NACC, EMB), f32),
    ],
)

NB = 2000
GRID = N // NB


def _write_m_chunks(hn, we, m_refs):
    for c, mref in enumerate(m_refs):
        mref[...] = lax.dot_general(hn, we[c * EMB:(c + 1) * EMB, :], _DN,
                                    precision=_PREC,
                                    preferred_element_type=f32)


def _tc_m0_body(h_ref, we_ref, m0_ref, m1_ref, m2_ref):
    _write_m_chunks(h_ref[...], we_ref[...], (m0_ref, m1_ref, m2_ref))


_m_out_shapes = [jax.ShapeDtypeStruct((N, EMB), f32) for _ in range(NCHUNK)]
_m_out_specs = [pl.BlockSpec((NB, EMB), lambda i: (i, 0)) for _ in range(NCHUNK)]

_tc_m0 = pl.pallas_call(
    _tc_m0_body,
    grid=(GRID,),
    in_specs=[
        pl.BlockSpec((NB, F), lambda i: (i, 0)),
        pl.BlockSpec((F, F), lambda i: (0, 0)),
    ],
    out_specs=_m_out_specs,
    out_shape=_m_out_shapes,
)


def _gru(h, p_ref, wih_ref, whh_ref, bih_ref, bhh_ref):
    p = p_ref[...]
    a = jnp.concatenate(
        [p[0, 0] + p[1, 0], p[0, 1] + p[1, 1], p[0, 2] + p[1, 2]], axis=1)
    gi = lax.dot_general(a, wih_ref[...], _DN, precision=_PREC,
                         preferred_element_type=f32) + bih_ref[...]
    gh = lax.dot_general(h, whh_ref[...], _DN, precision=_PREC,
                         preferred_element_type=f32) + bhh_ref[...]
    r = jax.nn.sigmoid(gi[:, :F] + gh[:, :F])
    z = jax.nn.sigmoid(gi[:, F:2 * F] + gh[:, F:2 * F])
    n = jnp.tanh(gi[:, 2 * F:] + r * gh[:, 2 * F:])
    return (1.0 - z) * n + z * h


def _tc_step_body(h_ref, p_ref, wih_ref, whh_ref, bih_ref, bhh_ref, we_ref,
                  hn_ref, m0_ref, m1_ref, m2_ref):
    hn = _gru(h_ref[...], p_ref, wih_ref, whh_ref, bih_ref, bhh_ref)
    hn_ref[...] = hn
    _write_m_chunks(hn, we_ref[...], (m0_ref, m1_ref, m2_ref))


_gru_in_specs = [
    pl.BlockSpec((NB, F), lambda i: (i, 0)),
    pl.BlockSpec((NC, NCHUNK, NB, EMB), lambda i: (0, 0, i, 0)),
    pl.BlockSpec((G3, F), lambda i: (0, 0)),
    pl.BlockSpec((G3, F), lambda i: (0, 0)),
    pl.BlockSpec((1, G3), lambda i: (0, 0)),
    pl.BlockSpec((1, G3), lambda i: (0, 0)),
]

_tc_step = pl.pallas_call(
    _tc_step_body,
    grid=(GRID,),
    in_specs=[
        *_gru_in_specs,
        pl.BlockSpec((F, F), lambda i: (0, 0)),
    ],
    out_specs=[
        pl.BlockSpec((NB, F), lambda i: (i, 0)),
        *_m_out_specs,
    ],
    out_shape=[
        jax.ShapeDtypeStruct((N, F), f32),
        *_m_out_shapes,
    ],
)


def _tc_final_body(h_ref, p_ref, wih_ref, whh_ref, bih_ref, bhh_ref,
                   h2_ref, h3_ref, cw_ref, cb_ref, out_ref):
    h4 = h_ref[...]
    h5 = _gru(h4, p_ref, wih_ref, whh_ref, bih_ref, bhh_ref)
    cw = cw_ref[...]
    cb = cb_ref[...]
    cs = [
        jax.nn.sigmoid(
            lax.dot_general(hh, cw, _DN, precision=_PREC,
                            preferred_element_type=f32) + cb)
        for hh in (h2_ref[...], h3_ref[...], h4, h5)
    ]
    out_ref[...] = jnp.stack(cs, axis=1).reshape(4 * NB, C2)


_tc_final = pl.pallas_call(
    _tc_final_body,
    grid=(GRID,),
    in_specs=[
        *_gru_in_specs,
        pl.BlockSpec((NB, F), lambda i: (i, 0)),
        pl.BlockSpec((NB, F), lambda i: (i, 0)),
        pl.BlockSpec((C2, F), lambda i: (0, 0)),
        pl.BlockSpec((1, C2), lambda i: (0, 0)),
    ],
    out_specs=pl.BlockSpec((4 * NB, C2), lambda i: (i, 0)),
    out_shape=jax.ShapeDtypeStruct((4 * N, C2), f32),
)


def kernel(prop_ids, edge_index, step_min, step_max, embed_table, W_e, w_ih,
           w_hh, b_ih, b_hh, conv_w, conv_b):
    del step_min, step_max
    idx = prop_ids.astype(i32).reshape(-1)
    idx = jnp.concatenate([idx, jnp.zeros((IDXP - N * NPROP,), i32)])
    idx2 = idx.reshape(IROWS, 128)
    emb = _embed_call(embed_table, idx2)
    h = emb[:N * NPROP].reshape(N, F)

    src = edge_index[0].astype(i32)
    dst = edge_index[1].astype(i32)
    pad_e = EP - E
    src_p = jnp.concatenate([src, jnp.zeros((pad_e,), i32)]).reshape(EROWS, 128)
    dst_pad = N + (jnp.arange(pad_e, dtype=i32) % (NACC - N))
    dst_p = jnp.concatenate([dst, dst_pad]).reshape(EROWS, 128)
    zeros_blk = jnp.zeros((STRIPE, EMB), f32)

    bih2 = b_ih.reshape(1, G3)
    bhh2 = b_hh.reshape(1, G3)
    cb2 = conv_b.reshape(1, C2)

    m0, m1, m2 = _tc_m0(h, W_e)
    hs = [h]
    for _ in range(NSTEP - 1):
        part = _scatter_call(src_p, dst_p, m0, m1, m2, zeros_blk)
        h, m0, m1, m2 = _tc_step(h, part, w_ih, w_hh, bih2, bhh2, W_e)
        hs.append(h)
    part = _scatter_call(src_p, dst_p, m0, m1, m2, zeros_blk)
    out = _tc_final(h, part, w_ih, w_hh, bih2, bhh2, hs[2], hs[3],
                    conv_w, cb2)
    return out
